# Initial kernel scaffold; baseline (speedup 1.0000x reference)
import jax
import jax.numpy as jnp
from jax.experimental import pallas as pl


def kernel(cls_logits, bbox_reg, anchors, gt_boxes):
    raise NotImplementedError("write your pallas kernel here")


# speedup vs baseline: 3.6295x; 3.6295x over previous
import functools

import jax
import jax.numpy as jnp
import numpy as np
from jax import lax
from jax.experimental import pallas as pl
from jax.experimental.pallas import tpu as pltpu

_FG = 0.7
_BG = 0.3
_K_POS = 128
_BATCH = 256
_LANES = 128
_NEG1_BITS = np.float32(-1.0).view(np.int32).item()
_ONE_BITS = np.float32(1.0).view(np.int32).item()
_FG_BITS = np.float32(_FG).view(np.int32).item()


def _kth_largest(keys, k, n_real):
    lo = jnp.int32(_FG_BITS - 1)
    hi = jnp.int32(_ONE_BITS + 1)

    def step(_, lohi):
        lo, hi = lohi
        mid = (lo + hi) >> 1
        big = jnp.sum((keys > mid).astype(jnp.int32)) >= k
        return jnp.where(big, mid, lo), jnp.where(big, hi, mid)

    _, hi = lax.fori_loop(0, 23, step, (lo, hi))
    return jnp.where(n_real >= k, hi, jnp.int32(_NEG1_BITS))


def _tie_bound(tie, idx, need, npad):
    lo = jnp.int32(0)
    hi = jnp.int32(npad)

    def step(_, lohi):
        lo, hi = lohi
        active = lo < hi
        mid = (lo + hi) >> 1
        ge = jnp.sum((tie & (idx < mid)).astype(jnp.int32)) >= need
        new_hi = jnp.where(ge, mid, hi)
        new_lo = jnp.where(ge, lo, mid + 1)
        return jnp.where(active, new_lo, lo), jnp.where(active, new_hi, hi)

    _, hi = lax.fori_loop(0, 15, step, (lo, hi))
    return hi


def _select_topk(pri, idx, k):
    keys = lax.bitcast_convert_type(pri, jnp.int32)
    n_real = jnp.sum((keys > jnp.int32(_NEG1_BITS)).astype(jnp.int32))
    tau = _kth_largest(keys, k, n_real)
    n_gt = jnp.sum((keys > tau).astype(jnp.int32))
    need = jnp.where(tau > jnp.int32(_NEG1_BITS), k - n_gt, 0)
    tie = keys == tau
    bound = _tie_bound(tie, idx, need, idx.size)
    sel = (keys > tau) | (tie & (idx < bound))
    return sel, n_gt + need


def _rpn_body(n_real, g_real, cl_ref, ax1_ref, ay1_ref, ax2_ref, ay2_ref,
              bb0_ref, bb1_ref, bb2_ref, bb3_ref, gt_ref, out_ref):
    ax1 = ax1_ref[0]
    ay1 = ay1_ref[0]
    ax2 = ax2_ref[0]
    ay2 = ay2_ref[0]
    area1 = (ax2 - ax1) * (ay2 - ay1)
    shape = ax1.shape

    neg_inf = jnp.full(shape, -jnp.inf, jnp.float32)
    zero = jnp.zeros(shape, jnp.float32)

    def gstep(g, carry):
        mx, b0, b1, b2, b3 = carry
        gx1 = gt_ref[0, 0, g]
        gy1 = gt_ref[0, 1, g]
        gx2 = gt_ref[0, 2, g]
        gy2 = gt_ref[0, 3, g]
        area2 = (gx2 - gx1) * (gy2 - gy1)
        w = jnp.maximum(jnp.minimum(ax2, gx2) - jnp.maximum(ax1, gx1), 0.0)
        h = jnp.maximum(jnp.minimum(ay2, gy2) - jnp.maximum(ay1, gy1), 0.0)
        inter = w * h
        iou = inter / ((area1 + area2) - inter)
        pred = iou > mx
        return (jnp.where(pred, iou, mx),
                jnp.where(pred, gx1, b0), jnp.where(pred, gy1, b1),
                jnp.where(pred, gx2, b2), jnp.where(pred, gy2, b3))

    mx, tx1, ty1, tx2, ty2 = lax.fori_loop(
        0, g_real, gstep, (neg_inf, zero, zero, zero, zero))

    idx = (lax.broadcasted_iota(jnp.int32, shape, 0) * _LANES
           + lax.broadcasted_iota(jnp.int32, shape, 1))
    valid_n = idx < n_real

    pos_pri = jnp.where(valid_n & (mx >= _FG), mx, -1.0)
    neg_pri = jnp.where(valid_n & (mx < _BG), 1.0 - mx, -1.0)

    pos_sel, num_pos = _select_topk(pos_pri, idx, _K_POS)
    neg_sel, num_neg = _select_topk(neg_pri, idx, _BATCH - num_pos)

    x = cl_ref[0]
    lab = pos_sel.astype(jnp.float32)
    bce = (jnp.maximum(x, 0.0) - x * lab
           + jnp.log(1.0 + jnp.exp(-jnp.abs(x))))
    cls_sum = jnp.sum(jnp.where(pos_sel | neg_sel, bce, 0.0))

    acx = (ax1 + ax2) / 2.0
    acy = (ay1 + ay2) / 2.0
    aw = ax2 - ax1
    ah = ay2 - ay1
    tcx = (tx1 + tx2) / 2.0
    tcy = (ty1 + ty2) / 2.0
    tw = tx2 - tx1
    th = ty2 - ty1

    d0 = bb0_ref[0] - (tcx - acx) / aw
    d1 = bb1_ref[0] - (tcy - acy) / ah
    d2 = bb2_ref[0] - jnp.log(tw / aw)
    d3 = bb3_ref[0] - jnp.log(th / ah)

    def sl1(d):
        ad = jnp.abs(d)
        return jnp.where(ad < 1.0, 0.5 * d * d, ad - 0.5)

    reg = sl1(d0) + sl1(d1) + sl1(d2) + sl1(d3)
    reg_sum = jnp.sum(jnp.where(pos_sel, reg, 0.0))

    num_pos_f = num_pos.astype(jnp.float32)
    lane = lax.broadcasted_iota(jnp.int32, (1, _LANES), 1)
    out = jnp.where(lane == 0, cls_sum,
          jnp.where(lane == 1, (num_pos + num_neg).astype(jnp.float32),
          jnp.where(lane == 2, reg_sum,
          jnp.where(lane == 3, 4.0 * num_pos_f, 0.0))))
    out_ref[0] = out


def kernel(cls_logits, bbox_reg, anchors, gt_boxes):
    b, n, _ = cls_logits.shape
    g = gt_boxes.shape[1]
    npad = ((n + 1023) // 1024) * 1024
    rows = npad // _LANES
    pad = npad - n

    def prep(x):
        return jnp.pad(x, ((0, 0), (0, pad))).reshape(b, rows, _LANES)

    cl = prep(cls_logits.reshape(b, n))
    planes = [prep(anchors[:, :, i]) for i in range(4)]
    planes += [prep(bbox_reg[:, :, i]) for i in range(4)]
    gt_t = jnp.transpose(gt_boxes, (0, 2, 1))

    vspec = pl.BlockSpec((1, rows, _LANES), lambda i: (i, 0, 0))
    gspec = pl.BlockSpec((1, 4, g), lambda i: (i, 0, 0),
                         memory_space=pltpu.SMEM)

    partials = pl.pallas_call(
        functools.partial(_rpn_body, n, g),
        grid=(b,),
        in_specs=[vspec] * 9 + [gspec],
        out_specs=pl.BlockSpec((1, 1, _LANES), lambda i: (i, 0, 0)),
        out_shape=jax.ShapeDtypeStruct((b, 1, _LANES), jnp.float32),
    )(cl, *planes, gt_t)

    sums = jnp.sum(partials[:, 0, :4], axis=0)
    cls_loss = sums[0] / jnp.maximum(sums[1], 1.0)
    reg_loss = jnp.where(sums[3] > 0.0,
                         sums[2] / jnp.maximum(sums[3], 1.0), 0.0)
    return jnp.stack([cls_loss, reg_loss])


# speedup vs baseline: 5.2685x; 1.4516x over previous
import functools

import jax
import jax.numpy as jnp
import numpy as np
from jax import lax
from jax.experimental import pallas as pl
from jax.experimental.pallas import tpu as pltpu

_FG = 0.7
_BG = 0.3
_K_POS = 128
_BATCH = 256
_LANES = 128
_SUB = 8
_NEG1_BITS = np.float32(-1.0).view(np.int32).item()
_ONE_BITS = np.float32(1.0).view(np.int32).item()
_FG_BITS = np.float32(_FG).view(np.int32).item()


def _dual_kth(pkeys, nkeys, kp, kn, np_real, nn_real):
    lo0 = jnp.int32(_FG_BITS - 1)
    hi0 = jnp.int32(_ONE_BITS + 1)

    def step(_, st):
        plo, phi, nlo, nhi = st
        pmid = (plo + phi) >> 1
        nmid = (nlo + nhi) >> 1
        pbig = jnp.sum((pkeys > pmid).astype(jnp.int32)) >= kp
        nbig = jnp.sum((nkeys > nmid).astype(jnp.int32)) >= kn
        return (jnp.where(pbig, pmid, plo), jnp.where(pbig, phi, pmid),
                jnp.where(nbig, nmid, nlo), jnp.where(nbig, nhi, nmid))

    _, phi, _, nhi = lax.fori_loop(0, 23, step, (lo0, hi0, lo0, hi0))
    tau_p = jnp.where(np_real >= kp, phi, jnp.int32(_NEG1_BITS))
    tau_n = jnp.where(nn_real >= kn, nhi, jnp.int32(_NEG1_BITS))
    return tau_p, tau_n


def _dual_tie_bound(ptie, ntie, idx, pneed, nneed, npad):
    z = jnp.int32(0)
    top = jnp.int32(npad)

    def step(_, st):
        plo, phi, nlo, nhi = st
        pact = plo < phi
        nact = nlo < nhi
        pmid = (plo + phi) >> 1
        nmid = (nlo + nhi) >> 1
        pge = jnp.sum((ptie & (idx < pmid)).astype(jnp.int32)) >= pneed
        nge = jnp.sum((ntie & (idx < nmid)).astype(jnp.int32)) >= nneed
        return (jnp.where(pact & ~pge, pmid + 1, plo),
                jnp.where(pact & pge, pmid, phi),
                jnp.where(nact & ~nge, nmid + 1, nlo),
                jnp.where(nact & nge, nmid, nhi))

    _, phi, _, nhi = lax.fori_loop(0, 15, step, (z, top, z, top))
    return phi, nhi


def _rpn_body(n_real, g_real, n_chunks, cl_ref, a0_ref, a1_ref, a2_ref,
              a3_ref, b0_ref, b1_ref, b2_ref, b3_ref, gt_ref, out_ref,
              mx_s, reg_s, com_s):
    def chunk(i, _):
        ax1 = a0_ref[0, i]
        ay1 = a1_ref[0, i]
        ax2 = a2_ref[0, i]
        ay2 = a3_ref[0, i]
        area1 = (ax2 - ax1) * (ay2 - ay1)
        shape = ax1.shape
        mx = jnp.full(shape, -jnp.inf, jnp.float32)
        tx1 = ty1 = tx2 = ty2 = jnp.zeros(shape, jnp.float32)
        for g in range(g_real):
            gx1 = gt_ref[0, 0, g]
            gy1 = gt_ref[0, 1, g]
            gx2 = gt_ref[0, 2, g]
            gy2 = gt_ref[0, 3, g]
            area2 = (gx2 - gx1) * (gy2 - gy1)
            w = jnp.maximum(jnp.minimum(ax2, gx2) - jnp.maximum(ax1, gx1), 0.0)
            h = jnp.maximum(jnp.minimum(ay2, gy2) - jnp.maximum(ay1, gy1), 0.0)
            inter = w * h
            iou = inter / ((area1 + area2) - inter)
            pred = iou > mx
            mx = jnp.where(pred, iou, mx)
            tx1 = jnp.where(pred, gx1, tx1)
            ty1 = jnp.where(pred, gy1, ty1)
            tx2 = jnp.where(pred, gx2, tx2)
            ty2 = jnp.where(pred, gy2, ty2)
        mx_s[i] = mx

        acx = (ax1 + ax2) / 2.0
        acy = (ay1 + ay2) / 2.0
        aw = ax2 - ax1
        ah = ay2 - ay1
        d0 = b0_ref[0, i] - ((tx1 + tx2) / 2.0 - acx) / aw
        d1 = b1_ref[0, i] - ((ty1 + ty2) / 2.0 - acy) / ah
        d2 = b2_ref[0, i] - jnp.log((tx2 - tx1) / aw)
        d3 = b3_ref[0, i] - jnp.log((ty2 - ty1) / ah)

        def sl1(d):
            ad = jnp.abs(d)
            return jnp.where(ad < 1.0, 0.5 * d * d, ad - 0.5)

        reg_s[i] = sl1(d0) + sl1(d1) + sl1(d2) + sl1(d3)
        x = cl_ref[0, i]
        com_s[i] = jnp.maximum(x, 0.0) + jnp.log(1.0 + jnp.exp(-jnp.abs(x)))
        return 0

    lax.fori_loop(0, n_chunks, chunk, 0)

    mx = mx_s[...]
    shape = mx.shape
    idx = (lax.broadcasted_iota(jnp.int32, shape, 0) * (_SUB * _LANES)
           + lax.broadcasted_iota(jnp.int32, shape, 1) * _LANES
           + lax.broadcasted_iota(jnp.int32, shape, 2))
    valid_n = idx < n_real

    pos_pri = jnp.where(valid_n & (mx >= _FG), mx, -1.0)
    neg_pri = jnp.where(valid_n & (mx < _BG), 1.0 - mx, -1.0)
    pkeys = lax.bitcast_convert_type(pos_pri, jnp.int32)
    nkeys = lax.bitcast_convert_type(neg_pri, jnp.int32)

    neg1 = jnp.int32(_NEG1_BITS)
    c_pos = jnp.sum((pkeys > neg1).astype(jnp.int32))
    c_neg = jnp.sum((nkeys > neg1).astype(jnp.int32))
    num_pos = jnp.minimum(jnp.int32(_K_POS), c_pos)
    kn = jnp.int32(_BATCH) - num_pos

    tau_p, tau_n = _dual_kth(pkeys, nkeys, jnp.int32(_K_POS), kn, c_pos, c_neg)
    n_gt_p = jnp.sum((pkeys > tau_p).astype(jnp.int32))
    n_gt_n = jnp.sum((nkeys > tau_n).astype(jnp.int32))
    need_p = jnp.where(tau_p > neg1, _K_POS - n_gt_p, 0)
    need_n = jnp.where(tau_n > neg1, kn - n_gt_n, 0)
    tie_p = pkeys == tau_p
    tie_n = nkeys == tau_n
    bound_p, bound_n = _dual_tie_bound(tie_p, tie_n, idx, need_p, need_n,
                                       idx.size)
    pos_sel = (pkeys > tau_p) | (tie_p & (idx < bound_p))
    neg_sel = (nkeys > tau_n) | (tie_n & (idx < bound_n))
    num_neg = n_gt_n + need_n

    com = com_s[...]
    x = cl_ref[0]
    cls_sum = (jnp.sum(jnp.where(pos_sel | neg_sel, com, 0.0))
               - jnp.sum(jnp.where(pos_sel, x, 0.0)))
    reg_sum = jnp.sum(jnp.where(pos_sel, reg_s[...], 0.0))

    lane = lax.broadcasted_iota(jnp.int32, (1, _LANES), 1)
    out = jnp.where(lane == 0, cls_sum,
          jnp.where(lane == 1, (num_pos + num_neg).astype(jnp.float32),
          jnp.where(lane == 2, reg_sum,
          jnp.where(lane == 3, 4.0 * num_pos.astype(jnp.float32), 0.0))))
    out_ref[0] = out


def kernel(cls_logits, bbox_reg, anchors, gt_boxes):
    b, n, _ = cls_logits.shape
    g = gt_boxes.shape[1]
    npad = -(-n // (_SUB * _LANES)) * (_SUB * _LANES)
    chunks = npad // (_SUB * _LANES)
    pad = npad - n

    def prep(x):
        return jnp.pad(x, ((0, 0), (0, pad))).reshape(b, chunks, _SUB, _LANES)

    cl = prep(cls_logits.reshape(b, n))
    planes = [prep(anchors[:, :, i]) for i in range(4)]
    planes += [prep(bbox_reg[:, :, i]) for i in range(4)]
    gt_t = jnp.transpose(gt_boxes, (0, 2, 1))

    vspec = pl.BlockSpec((1, chunks, _SUB, _LANES), lambda i: (i, 0, 0, 0))
    gspec = pl.BlockSpec((1, 4, g), lambda i: (i, 0, 0),
                         memory_space=pltpu.SMEM)

    partials = pl.pallas_call(
        functools.partial(_rpn_body, n, g, chunks),
        grid=(b,),
        in_specs=[vspec] * 9 + [gspec],
        out_specs=pl.BlockSpec((1, 1, _LANES), lambda i: (i, 0, 0)),
        out_shape=jax.ShapeDtypeStruct((b, 1, _LANES), jnp.float32),
        scratch_shapes=[pltpu.VMEM((chunks, _SUB, _LANES), jnp.float32)] * 3,
    )(cl, *planes, gt_t)

    sums = jnp.sum(partials[:, 0, :4], axis=0)
    cls_loss = sums[0] / jnp.maximum(sums[1], 1.0)
    reg_loss = jnp.where(sums[3] > 0.0,
                         sums[2] / jnp.maximum(sums[3], 1.0), 0.0)
    return jnp.stack([cls_loss, reg_loss])


# speedup vs baseline: 5.4610x; 1.0365x over previous
import functools

import jax
import jax.numpy as jnp
import numpy as np
from jax import lax
from jax.experimental import pallas as pl
from jax.experimental.pallas import tpu as pltpu

_FG = 0.7
_BG = 0.3
_K_POS = 128
_BATCH = 256
_LANES = 128
_SUB = 16
_NEG1_BITS = np.float32(-1.0).view(np.int32).item()
_ONE_BITS = np.float32(1.0).view(np.int32).item()
_FG_BITS = np.float32(_FG).view(np.int32).item()


def _dual_kth(pkeys, nkeys, kp, kn, np_real, nn_real):
    lo0 = jnp.int32(_FG_BITS - 1)
    hi0 = jnp.int32(_ONE_BITS + 1)

    def step(_, st):
        plo, phi, nlo, nhi = st
        pmid = (plo + phi) >> 1
        nmid = (nlo + nhi) >> 1
        pbig = jnp.sum((pkeys > pmid).astype(jnp.int32)) >= kp
        nbig = jnp.sum((nkeys > nmid).astype(jnp.int32)) >= kn
        return (jnp.where(pbig, pmid, plo), jnp.where(pbig, phi, pmid),
                jnp.where(nbig, nmid, nlo), jnp.where(nbig, nhi, nmid))

    _, phi, _, nhi = lax.fori_loop(0, 23, step, (lo0, hi0, lo0, hi0))
    tau_p = jnp.where(np_real >= kp, phi, jnp.int32(_NEG1_BITS))
    tau_n = jnp.where(nn_real >= kn, nhi, jnp.int32(_NEG1_BITS))
    return tau_p, tau_n


def _dual_tie_bound(ptie, ntie, idx, pneed, nneed, npad):
    z = jnp.int32(0)
    top = jnp.int32(npad)

    def step(_, st):
        plo, phi, nlo, nhi = st
        pact = plo < phi
        nact = nlo < nhi
        pmid = (plo + phi) >> 1
        nmid = (nlo + nhi) >> 1
        pge = jnp.sum((ptie & (idx < pmid)).astype(jnp.int32)) >= pneed
        nge = jnp.sum((ntie & (idx < nmid)).astype(jnp.int32)) >= nneed
        return (jnp.where(pact & ~pge, pmid + 1, plo),
                jnp.where(pact & pge, pmid, phi),
                jnp.where(nact & ~nge, nmid + 1, nlo),
                jnp.where(nact & nge, nmid, nhi))

    _, phi, _, nhi = lax.fori_loop(0, 15, step, (z, top, z, top))
    return phi, nhi


def _rpn_body(n_real, g_real, n_chunks, cl_ref, a0_ref, a1_ref, a2_ref,
              a3_ref, b0_ref, b1_ref, b2_ref, b3_ref, gt_ref, out_ref,
              mx_s, reg_s, com_s):
    def chunk(i, _):
        ax1 = a0_ref[0, i]
        ay1 = a1_ref[0, i]
        ax2 = a2_ref[0, i]
        ay2 = a3_ref[0, i]
        area1 = (ax2 - ax1) * (ay2 - ay1)
        shape = ax1.shape
        neg_inf = jnp.full(shape, -jnp.inf, jnp.float32)
        zero = jnp.zeros(shape, jnp.float32)

        n_chains = 4
        per = -(-g_real // n_chains)
        chains = []
        for c in range(n_chains):
            mx = neg_inf
            tx1 = ty1 = tx2 = ty2 = zero
            for g in range(c * per, min((c + 1) * per, g_real)):
                gx1 = gt_ref[0, 0, g]
                gy1 = gt_ref[0, 1, g]
                gx2 = gt_ref[0, 2, g]
                gy2 = gt_ref[0, 3, g]
                area2 = (gx2 - gx1) * (gy2 - gy1)
                w = jnp.maximum(
                    jnp.minimum(ax2, gx2) - jnp.maximum(ax1, gx1), 0.0)
                h = jnp.maximum(
                    jnp.minimum(ay2, gy2) - jnp.maximum(ay1, gy1), 0.0)
                inter = w * h
                iou = inter / ((area1 + area2) - inter)
                pred = iou > mx
                mx = jnp.where(pred, iou, mx)
                tx1 = jnp.where(pred, gx1, tx1)
                ty1 = jnp.where(pred, gy1, ty1)
                tx2 = jnp.where(pred, gx2, tx2)
                ty2 = jnp.where(pred, gy2, ty2)
            chains.append((mx, tx1, ty1, tx2, ty2))

        def merge(a, b):
            pred = b[0] > a[0]
            return tuple(jnp.where(pred, bb, aa) for aa, bb in zip(a, b))

        mx, tx1, ty1, tx2, ty2 = merge(merge(chains[0], chains[1]),
                                       merge(chains[2], chains[3]))
        mx_s[i] = mx

        acx = (ax1 + ax2) / 2.0
        acy = (ay1 + ay2) / 2.0
        aw = ax2 - ax1
        ah = ay2 - ay1
        d0 = b0_ref[0, i] - ((tx1 + tx2) / 2.0 - acx) / aw
        d1 = b1_ref[0, i] - ((ty1 + ty2) / 2.0 - acy) / ah
        d2 = b2_ref[0, i] - jnp.log((tx2 - tx1) / aw)
        d3 = b3_ref[0, i] - jnp.log((ty2 - ty1) / ah)

        def sl1(d):
            ad = jnp.abs(d)
            return jnp.where(ad < 1.0, 0.5 * d * d, ad - 0.5)

        reg_s[i] = sl1(d0) + sl1(d1) + sl1(d2) + sl1(d3)
        x = cl_ref[0, i]
        com_s[i] = jnp.maximum(x, 0.0) + jnp.log(1.0 + jnp.exp(-jnp.abs(x)))
        return 0

    lax.fori_loop(0, n_chunks, chunk, 0)

    mx = mx_s[...]
    shape = mx.shape
    idx = (lax.broadcasted_iota(jnp.int32, shape, 0) * (_SUB * _LANES)
           + lax.broadcasted_iota(jnp.int32, shape, 1) * _LANES
           + lax.broadcasted_iota(jnp.int32, shape, 2))
    valid_n = idx < n_real

    pos_pri = jnp.where(valid_n & (mx >= _FG), mx, -1.0)
    neg_pri = jnp.where(valid_n & (mx < _BG), 1.0 - mx, -1.0)
    pkeys = lax.bitcast_convert_type(pos_pri, jnp.int32)
    nkeys = lax.bitcast_convert_type(neg_pri, jnp.int32)

    neg1 = jnp.int32(_NEG1_BITS)
    c_pos = jnp.sum((pkeys > neg1).astype(jnp.int32))
    c_neg = jnp.sum((nkeys > neg1).astype(jnp.int32))
    num_pos = jnp.minimum(jnp.int32(_K_POS), c_pos)
    kn = jnp.int32(_BATCH) - num_pos

    tau_p, tau_n = _dual_kth(pkeys, nkeys, jnp.int32(_K_POS), kn, c_pos, c_neg)
    n_gt_p = jnp.sum((pkeys > tau_p).astype(jnp.int32))
    n_gt_n = jnp.sum((nkeys > tau_n).astype(jnp.int32))
    need_p = jnp.where(tau_p > neg1, _K_POS - n_gt_p, 0)
    need_n = jnp.where(tau_n > neg1, kn - n_gt_n, 0)
    tie_p = pkeys == tau_p
    tie_n = nkeys == tau_n
    bound_p, bound_n = _dual_tie_bound(tie_p, tie_n, idx, need_p, need_n,
                                       idx.size)
    pos_sel = (pkeys > tau_p) | (tie_p & (idx < bound_p))
    neg_sel = (nkeys > tau_n) | (tie_n & (idx < bound_n))
    num_neg = n_gt_n + need_n

    com = com_s[...]
    x = cl_ref[0]
    cls_sum = (jnp.sum(jnp.where(pos_sel | neg_sel, com, 0.0))
               - jnp.sum(jnp.where(pos_sel, x, 0.0)))
    reg_sum = jnp.sum(jnp.where(pos_sel, reg_s[...], 0.0))

    lane = lax.broadcasted_iota(jnp.int32, (1, _LANES), 1)
    out = jnp.where(lane == 0, cls_sum,
          jnp.where(lane == 1, (num_pos + num_neg).astype(jnp.float32),
          jnp.where(lane == 2, reg_sum,
          jnp.where(lane == 3, 4.0 * num_pos.astype(jnp.float32), 0.0))))
    out_ref[0] = out


def kernel(cls_logits, bbox_reg, anchors, gt_boxes):
    b, n, _ = cls_logits.shape
    g = gt_boxes.shape[1]
    npad = -(-n // (_SUB * _LANES)) * (_SUB * _LANES)
    chunks = npad // (_SUB * _LANES)
    pad = npad - n

    def prep(x):
        return jnp.pad(x, ((0, 0), (0, pad))).reshape(b, chunks, _SUB, _LANES)

    cl = prep(cls_logits.reshape(b, n))
    planes = [prep(anchors[:, :, i]) for i in range(4)]
    planes += [prep(bbox_reg[:, :, i]) for i in range(4)]
    gt_t = jnp.transpose(gt_boxes, (0, 2, 1))

    vspec = pl.BlockSpec((1, chunks, _SUB, _LANES), lambda i: (i, 0, 0, 0))
    gspec = pl.BlockSpec((1, 4, g), lambda i: (i, 0, 0),
                         memory_space=pltpu.SMEM)

    partials = pl.pallas_call(
        functools.partial(_rpn_body, n, g, chunks),
        grid=(b,),
        in_specs=[vspec] * 9 + [gspec],
        out_specs=pl.BlockSpec((1, 1, _LANES), lambda i: (i, 0, 0)),
        out_shape=jax.ShapeDtypeStruct((b, 1, _LANES), jnp.float32),
        scratch_shapes=[pltpu.VMEM((chunks, _SUB, _LANES), jnp.float32)] * 3,
    )(cl, *planes, gt_t)

    sums = jnp.sum(partials[:, 0, :4], axis=0)
    cls_loss = sums[0] / jnp.maximum(sums[1], 1.0)
    reg_loss = jnp.where(sums[3] > 0.0,
                         sums[2] / jnp.maximum(sums[3], 1.0), 0.0)
    return jnp.stack([cls_loss, reg_loss])
